# Initial kernel scaffold; baseline (speedup 1.0000x reference)
import jax
import jax.numpy as jnp
from jax.experimental import pallas as pl


def kernel(gmap, atom, bonds, W_be, b_be, W_ae, b_ae, W_bu, b_bu, W_au, b_au, W_fc, b_fc):
    raise NotImplementedError("write your pallas kernel here")


# speedup vs baseline: 2.8526x; 2.8526x over previous
import functools

import jax
import jax.numpy as jnp
from jax import lax
from jax.experimental import pallas as pl
from jax.experimental.pallas import tpu as pltpu
from jax.experimental.pallas import tpu_sc as plsc

B = 10000
NNN = 32
NAF = 128
NBF = 16
H1 = 128
H2 = 128
NCONV = 3
E = B * NNN

_NC = 2
_NS = 16
_NW = _NC * _NS
_BPW = E // _NW
_CH = 200
_NCH = _BPW // _CH

@functools.cache
def _make_sc_gather():
    mesh = plsc.VectorSubcoreMesh(
        core_axis_name="c", subcore_axis_name="s",
        num_cores=_NC, num_subcores=_NS,
    )

    @functools.partial(
        pl.kernel,
        out_type=jax.ShapeDtypeStruct((E, H1), jnp.float32),
        mesh=mesh,
        scratch_types=[
            pltpu.VMEM((_BPW,), jnp.int32),
            pltpu.VMEM((2, _CH, H1), jnp.float32),
            pltpu.SemaphoreType.DMA,
            pltpu.SemaphoreType.DMA,
        ],
    )
    def sc_gather(table_hbm, idx_hbm, out_hbm, idx_v, rows_v, sem0, sem1):
        wid = lax.axis_index("s") * _NC + lax.axis_index("c")
        base = wid * _BPW
        pltpu.sync_copy(idx_hbm.at[pl.ds(base, _BPW)], idx_v)
        sems = (sem0, sem1)
        for b in range(2):
            pltpu.async_copy(
                table_hbm.at[idx_v.at[pl.ds(b * _CH, _CH)]], rows_v.at[b], sems[b]
            )

        @pl.loop(0, _NCH, step=2)
        def _(k):
            for b in range(2):
                cur = k + b
                pltpu.make_async_copy(
                    table_hbm.at[idx_v.at[pl.ds(cur * _CH, _CH)]],
                    rows_v.at[b],
                    sems[b],
                ).wait()
                pltpu.sync_copy(
                    rows_v.at[b], out_hbm.at[pl.ds(base + cur * _CH, _CH)]
                )
                nxt = cur + 2

                @pl.when(nxt < _NCH)
                def _():
                    pltpu.async_copy(
                        table_hbm.at[idx_v.at[pl.ds(nxt * _CH, _CH)]],
                        rows_v.at[b],
                        sems[b],
                    )

    return sc_gather


def _sc_gather(table, idx):
    return _make_sc_gather()(table, idx)


def _softplus(x):
    return jnp.maximum(x, 0.0) + jnp.log1p(jnp.exp(-jnp.abs(x)))


_F32 = jnp.float32


def _dot(a, b):
    return jnp.dot(a, b, preferred_element_type=_F32)


_PM = 2000


def _pre_body(atom_ref, ws_ref, wn_ref, bias_ref, a_ref, g_ref):
    x = atom_ref[...]
    a_ref[...] = _dot(x, ws_ref[...]) + bias_ref[...]
    g_ref[...] = _dot(x, wn_ref[...])


def _pre(atom, w_self, w_nbr, bias2d):
    w_spec = pl.BlockSpec((NAF, H1), lambda i: (0, 0))
    return pl.pallas_call(
        _pre_body,
        grid=(B // _PM,),
        in_specs=[
            pl.BlockSpec((_PM, NAF), lambda i: (i, 0)),
            w_spec,
            w_spec,
            pl.BlockSpec((1, H1), lambda i: (0, 0)),
        ],
        out_specs=[
            pl.BlockSpec((_PM, H1), lambda i: (i, 0)),
            pl.BlockSpec((_PM, H1), lambda i: (i, 0)),
        ],
        out_shape=[
            jax.ShapeDtypeStruct((B, H1), _F32),
            jax.ShapeDtypeStruct((B, H1), _F32),
        ],
    )(atom, w_self, w_nbr, bias2d)


_BM = 400


def _round_body(bonds_ref, ng_ref, a_ref, ah_ref, wb_ref, wam_ref, waa_ref,
                ba_ref, ws_ref, wn_ref, bn_ref, wfc_ref, bfc_ref,
                bh_out, ah_out, a_out, g_out, y_out):
    kdim = bonds_ref.shape[2]
    x = bonds_ref[...].reshape(_BM * NNN, kdim)
    c = _dot(x, wb_ref[...]).reshape(_BM, NNN, H1)
    t = jnp.tanh(a_ref[...][:, None, :] + ng_ref[...] + c)
    bh_out[...] = t
    m = jnp.mean(t, axis=1)
    ah = jnp.maximum(
        _dot(m, wam_ref[...]) + _dot(ah_ref[...], waa_ref[...]) + ba_ref[...], 0.0
    )
    ah_out[...] = ah
    a_out[...] = _dot(ah, ws_ref[...]) + bn_ref[...]
    g_out[...] = _dot(ah, wn_ref[...])
    y_out[...] = _softplus(_dot(ah, wfc_ref[...]) + bfc_ref[...])[:, 0:1]


def _round(bonds_in, ng, a, ah, w_bond, w_am, w_aa, ba2, w_self, w_nbr, bn2,
           wfc_pad, bfc2):
    kdim = bonds_in.shape[2]
    w128 = pl.BlockSpec((H1, H1), lambda i: (0, 0))
    b128 = pl.BlockSpec((1, H1), lambda i: (0, 0))
    row = pl.BlockSpec((_BM, H1), lambda i: (i, 0))
    return pl.pallas_call(
        _round_body,
        grid=(B // _BM,),
        in_specs=[
            pl.BlockSpec((_BM, NNN, kdim), lambda i: (i, 0, 0)),
            pl.BlockSpec((_BM, NNN, H1), lambda i: (i, 0, 0)),
            row,
            row,
            pl.BlockSpec((kdim, H1), lambda i: (0, 0)),
            w128, w128, b128, w128, w128, b128,
            w128, b128,
        ],
        out_specs=[
            pl.BlockSpec((_BM, NNN, H1), lambda i: (i, 0, 0)),
            row, row, row,
            pl.BlockSpec((_BM, 1), lambda i: (i, 0)),
        ],
        out_shape=[
            jax.ShapeDtypeStruct((B, NNN, H1), _F32),
            jax.ShapeDtypeStruct((B, H1), _F32),
            jax.ShapeDtypeStruct((B, H1), _F32),
            jax.ShapeDtypeStruct((B, H1), _F32),
            jax.ShapeDtypeStruct((B, 1), _F32),
        ],
    )(bonds_in, ng, a, ah, w_bond, w_am, w_aa, ba2, w_self, w_nbr, bn2,
      wfc_pad, bfc2)


def kernel(gmap, atom, bonds, W_be, b_be, W_ae, b_ae, W_bu, b_bu, W_au, b_au,
           W_fc, b_fc):
    idx = gmap.astype(jnp.int32).reshape(E)

    wbe_s, wbe_n, wbe_b = W_be[:NAF], W_be[NAF:2 * NAF], W_be[2 * NAF:]
    wae_m, wae_a = W_ae[:H1], W_ae[H1:]
    wbu_s, wbu_n, wbu_b = W_bu[:H2], W_bu[H2:2 * H2], W_bu[2 * H2:]
    wau_m, wau_a = W_au[:H1], W_au[H1:]
    b_be2 = b_be.reshape(1, H1)
    b_ae2 = b_ae.reshape(1, H2)
    b_bu2 = b_bu.reshape(1, H1)
    b_au2 = b_au.reshape(1, H2)
    wfc_pad = jnp.zeros((H2, H1), _F32).at[:, 0:1].set(W_fc)
    b_fc2 = jnp.broadcast_to(b_fc.reshape(1, 1), (1, H1))

    a0, g0 = _pre(atom, wbe_s, wbe_n, b_be2)
    ng = _sc_gather(g0, idx).reshape(B, NNN, H1)
    bh, ah, a, g, y = _round(
        bonds, ng, a0, atom, wbe_b, wae_m, wae_a, b_ae2, wbu_s, wbu_n, b_bu2,
        wfc_pad, b_fc2,
    )

    for _ in range(NCONV):
        ng = _sc_gather(g, idx).reshape(B, NNN, H1)
        bh, ah, a, g, y = _round(
            bh, ng, a, ah, wbu_b, wau_m, wau_a, b_au2, wbu_s, wbu_n, b_bu2,
            wfc_pad, b_fc2,
        )
    return y


# speedup vs baseline: 3.3162x; 1.1625x over previous
import functools

import jax
import jax.numpy as jnp
from jax import lax
from jax.experimental import pallas as pl
from jax.experimental.pallas import tpu as pltpu
from jax.experimental.pallas import tpu_sc as plsc

B = 10000
NNN = 32
NAF = 128
NBF = 16
H1 = 128
H2 = 128
NCONV = 3
E = B * NNN
HP = H1 // 2

_NC = 2
_NS = 16
_NW = _NC * _NS
_BPW = E // _NW
_CH = 200
_NCH = _BPW // _CH

_F32 = jnp.float32
_BF16 = jnp.bfloat16
_U32 = jnp.uint32


@functools.cache
def _make_sc_gather():
    mesh = plsc.VectorSubcoreMesh(
        core_axis_name="c", subcore_axis_name="s",
        num_cores=_NC, num_subcores=_NS,
    )

    @functools.partial(
        pl.kernel,
        out_type=jax.ShapeDtypeStruct((E, H1), _F32),
        mesh=mesh,
        scratch_types=[
            pltpu.VMEM((_BPW,), jnp.int32),
            pltpu.VMEM((2, _CH, H1), _F32),
            pltpu.SemaphoreType.DMA,
            pltpu.SemaphoreType.DMA,
        ],
    )
    def sc_gather(table_hbm, idx_hbm, out_hbm, idx_v, rows_v, sem0, sem1):
        wid = lax.axis_index("s") * _NC + lax.axis_index("c")
        base = wid * _BPW
        pltpu.sync_copy(idx_hbm.at[pl.ds(base, _BPW)], idx_v)
        sems = (sem0, sem1)
        for b in range(2):
            pltpu.async_copy(
                table_hbm.at[idx_v.at[pl.ds(b * _CH, _CH)]], rows_v.at[b], sems[b]
            )

        @pl.loop(0, _NCH, step=2)
        def _(k):
            for b in range(2):
                cur = k + b
                pltpu.make_async_copy(
                    table_hbm.at[idx_v.at[pl.ds(cur * _CH, _CH)]],
                    rows_v.at[b],
                    sems[b],
                ).wait()
                pltpu.sync_copy(
                    rows_v.at[b], out_hbm.at[pl.ds(base + cur * _CH, _CH)]
                )
                nxt = cur + 2

                @pl.when(nxt < _NCH)
                def _():
                    pltpu.async_copy(
                        table_hbm.at[idx_v.at[pl.ds(nxt * _CH, _CH)]],
                        rows_v.at[b],
                        sems[b],
                    )

    return sc_gather


def _gather_f32(g, idx):
    return _make_sc_gather()(g, idx).reshape(B, NNN, H1)


def _softplus(x):
    return jnp.maximum(x, 0.0) + jnp.log1p(jnp.exp(-jnp.abs(x)))


def _dot(a, b):
    return jnp.dot(a, b, preferred_element_type=_F32)


_PM = 2000


def _pre_body(atom_ref, ws_ref, wn_ref, bias_ref, a_ref, g_ref):
    x = atom_ref[...]
    a_ref[...] = _dot(x, ws_ref[...]) + bias_ref[...]
    g_ref[...] = _dot(x, wn_ref[...])


def _pre(atom, w_self, w_nbr, bias2d):
    w_spec = pl.BlockSpec((NAF, H1), lambda i: (0, 0))
    return pl.pallas_call(
        _pre_body,
        grid=(B // _PM,),
        in_specs=[
            pl.BlockSpec((_PM, NAF), lambda i: (i, 0)),
            w_spec,
            w_spec,
            pl.BlockSpec((1, H1), lambda i: (0, 0)),
        ],
        out_specs=[
            pl.BlockSpec((_PM, H1), lambda i: (i, 0)),
            pl.BlockSpec((_PM, H1), lambda i: (i, 0)),
        ],
        out_shape=[
            jax.ShapeDtypeStruct((B, H1), _F32),
            jax.ShapeDtypeStruct((B, H1), _F32),
        ],
    )(atom, w_self, w_nbr, bias2d)


_BM = 400


def _round_body(bonds_ref, ng_ref, a_ref, ah_ref, wb_ref, wam_ref, waa_ref,
                ba_ref, ws_ref, wn_ref, bn_ref, wfc_ref, bfc_ref,
                bh_out, ah_out, a_out, g_out, y_out):
    kdim = bonds_ref.shape[2]
    x = bonds_ref[...].reshape(_BM * NNN, kdim)
    c = _dot(x, wb_ref[...]).reshape(_BM, NNN, H1)
    t = jnp.tanh(a_ref[...][:, None, :] + ng_ref[...] + c)
    bh_out[...] = t.astype(_BF16)
    m = jnp.mean(t, axis=1)
    ah = jnp.maximum(
        _dot(m, wam_ref[...]) + _dot(ah_ref[...], waa_ref[...]) + ba_ref[...], 0.0
    )
    ah_out[...] = ah
    a_out[...] = _dot(ah, ws_ref[...]) + bn_ref[...]
    g_out[...] = _dot(ah, wn_ref[...])
    y_out[...] = _softplus(_dot(ah, wfc_ref[...]) + bfc_ref[...])[:, 0:1]


def _round(bonds_in, ng, a, ah, w_bond, w_am, w_aa, ba2, w_self, w_nbr, bn2,
           wfc_pad, bfc2):
    kdim = bonds_in.shape[2]
    w128 = pl.BlockSpec((H1, H1), lambda i: (0, 0))
    b128 = pl.BlockSpec((1, H1), lambda i: (0, 0))
    row = pl.BlockSpec((_BM, H1), lambda i: (i, 0))
    return pl.pallas_call(
        _round_body,
        grid=(B // _BM,),
        in_specs=[
            pl.BlockSpec((_BM, NNN, kdim), lambda i: (i, 0, 0)),
            pl.BlockSpec((_BM, NNN, H1), lambda i: (i, 0, 0)),
            row,
            row,
            pl.BlockSpec((kdim, H1), lambda i: (0, 0)),
            w128, w128, b128, w128, w128, b128,
            w128, b128,
        ],
        out_specs=[
            pl.BlockSpec((_BM, NNN, H1), lambda i: (i, 0, 0)),
            row, row, row,
            pl.BlockSpec((_BM, 1), lambda i: (i, 0)),
        ],
        out_shape=[
            jax.ShapeDtypeStruct((B, NNN, H1), _BF16),
            jax.ShapeDtypeStruct((B, H1), _F32),
            jax.ShapeDtypeStruct((B, H1), _F32),
            jax.ShapeDtypeStruct((B, H1), _F32),
            jax.ShapeDtypeStruct((B, 1), _F32),
        ],
    )(bonds_in, ng, a, ah, w_bond, w_am, w_aa, ba2, w_self, w_nbr, bn2,
      wfc_pad, bfc2)


def kernel(gmap, atom, bonds, W_be, b_be, W_ae, b_ae, W_bu, b_bu, W_au, b_au,
           W_fc, b_fc):
    idx = gmap.astype(jnp.int32).reshape(E)

    wbe_s, wbe_n, wbe_b = W_be[:NAF], W_be[NAF:2 * NAF], W_be[2 * NAF:]
    wae_m, wae_a = W_ae[:H1], W_ae[H1:]
    wbu_s, wbu_n, wbu_b = W_bu[:H2], W_bu[H2:2 * H2], W_bu[2 * H2:]
    wau_m, wau_a = W_au[:H1], W_au[H1:]
    b_be2 = b_be.reshape(1, H1)
    b_ae2 = b_ae.reshape(1, H2)
    b_bu2 = b_bu.reshape(1, H1)
    b_au2 = b_au.reshape(1, H2)
    wbu_b16 = wbu_b.astype(_BF16)
    wfc_pad = jnp.zeros((H2, H1), _F32).at[:, 0:1].set(W_fc)
    b_fc2 = jnp.broadcast_to(b_fc.reshape(1, 1), (1, H1))

    a0, g0 = _pre(atom, wbe_s, wbe_n, b_be2)
    ng = _gather_f32(g0, idx)
    bh, ah, a, g, y = _round(
        bonds, ng, a0, atom, wbe_b, wae_m, wae_a, b_ae2, wbu_s, wbu_n, b_bu2,
        wfc_pad, b_fc2,
    )

    for _ in range(NCONV):
        ng = _gather_f32(g, idx)
        bh, ah, a, g, y = _round(
            bh, ng, a, ah, wbu_b16, wau_m, wau_a, b_au2, wbu_s, wbu_n, b_bu2,
            wfc_pad, b_fc2,
        )
    return y
